# baseline (speedup 1.0000x reference)
import jax
import jax.numpy as jnp
from jax.experimental import pallas as pl
from jax.experimental.pallas import tpu as pltpu


def _copy_body(x_ref, o_ref, sem):
    copy = pltpu.make_async_copy(x_ref, o_ref, sem)
    copy.start()
    copy.wait()


def kernel(inputs_embeds):
    out = pl.pallas_call(
        _copy_body,
        out_shape=jax.ShapeDtypeStruct(inputs_embeds.shape, inputs_embeds.dtype),
        in_specs=[pl.BlockSpec(memory_space=pl.ANY)],
        out_specs=pl.BlockSpec(memory_space=pl.ANY),
        scratch_shapes=[pltpu.SemaphoreType.DMA],
    )(inputs_embeds)
    return (out, jnp.float32(0.0))


# speedup vs baseline: 41.5659x; 41.5659x over previous
import jax
import jax.numpy as jnp
from jax.experimental import pallas as pl
from jax.experimental.pallas import tpu as pltpu


_BLOCK_ROWS = 2048


def _copy_body(x_ref, o_ref):
    o_ref[...] = x_ref[...]


def kernel(inputs_embeds):
    shape = inputs_embeds.shape
    x2d = inputs_embeds.reshape(-1, shape[-1])
    rows, cols = x2d.shape
    grid = (rows // _BLOCK_ROWS,)
    out = pl.pallas_call(
        _copy_body,
        out_shape=jax.ShapeDtypeStruct((rows, cols), x2d.dtype),
        grid=grid,
        in_specs=[pl.BlockSpec((_BLOCK_ROWS, cols), lambda i: (i, 0))],
        out_specs=pl.BlockSpec((_BLOCK_ROWS, cols), lambda i: (i, 0)),
    )(x2d)
    return (out.reshape(shape), jnp.float32(0.0))
